# baseline (speedup 1.0000x reference)
import jax
import jax.numpy as jnp
from jax.experimental import pallas as pl
from jax.experimental.pallas import tpu as pltpu

NUM_EMBEDDINGS = 512
EMBEDDING_DIM = 64
COMMITMENT_COST = 0.25
N_ROWS = 16 * 32 * 32
BLOCK_ROWS = 4096
GRID = N_ROWS // BLOCK_ROWS
GRID_IMG = 16 // GRID


def _vq_body(x_ref, xc_ref, e_ref, enc_ref, qst_ref, loss_ref, perp_ref,
             sse_acc, cnt_acc, se_scr):
    i = pl.program_id(0)
    x = x_ref[...]
    e = e_ref[...]

    @pl.when(i == 0)
    def _init():
        sse_acc[0, 0] = 0.0
        cnt_acc[...] = jnp.zeros_like(cnt_acc)

    sx = jnp.sum(x * x, axis=1, keepdims=True)
    se = jnp.sum(e * e, axis=1)
    mm2 = jax.lax.dot_general(x, e + e, (((1,), (1,)), ((), ())))

    c_iota = jax.lax.broadcasted_iota(
        jnp.int32, (mm2.shape[0], 128), 1).astype(jnp.float32)
    best = sx + se[0:128] - mm2[:, 0:128]
    bidx = c_iota
    for v in range(1, 4):
        dv = sx + se[v * 128:(v + 1) * 128] - mm2[:, v * 128:(v + 1) * 128]
        mask = dv < best
        best = jnp.where(mask, dv, best)
        bidx = jnp.where(mask, c_iota + float(v * 128), bidx)
    m = jnp.min(best, axis=1, keepdims=True)
    idxf = jnp.min(jnp.where(best == m, bidx, float(NUM_EMBEDDINGS)), axis=1,
                   keepdims=True)
    idx = idxf.astype(jnp.int32)
    iota = jax.lax.broadcasted_iota(jnp.int32, (mm2.shape[0], NUM_EMBEDDINGS), 1)
    enc = (iota == idx).astype(jnp.float32)
    enc_ref[...] = enc

    quant = jnp.dot(enc, e)
    diff = quant - x
    qst_ref[...] = x + diff

    sse_acc[0, 0] += jnp.sum(diff * diff)
    cnt_acc[...] += jnp.sum(enc, axis=0, keepdims=True)

    @pl.when(i == GRID - 1)
    def _fini():
        mean = sse_acc[0, 0] / float(N_ROWS * EMBEDDING_DIM)
        loss_ref[...] = jnp.full((1, 1), mean + COMMITMENT_COST * mean,
                                 dtype=jnp.float32)
        p = cnt_acc[...] / float(N_ROWS)
        ent = -jnp.sum(p * jnp.log(p + 1e-10))
        perp_ref[...] = jnp.exp(jnp.full((1, 1), ent, dtype=jnp.float32))


def kernel(inputs, labels, embedding):
    del labels
    x = jnp.transpose(inputs, (0, 2, 3, 1))
    input_shape = x.shape
    flat = x.reshape(-1, EMBEDDING_DIM)

    enc, qst, loss, perp = pl.pallas_call(
        _vq_body,
        grid=(GRID,),
        in_specs=[
            pl.BlockSpec((BLOCK_ROWS, EMBEDDING_DIM), lambda i: (i, 0)),
            pl.BlockSpec((GRID_IMG, EMBEDDING_DIM, 1024), lambda i: (i, 0, 0)),
            pl.BlockSpec((NUM_EMBEDDINGS, EMBEDDING_DIM), lambda i: (0, 0)),
        ],
        out_specs=[
            pl.BlockSpec((BLOCK_ROWS, NUM_EMBEDDINGS), lambda i: (i, 0)),
            pl.BlockSpec((BLOCK_ROWS, EMBEDDING_DIM), lambda i: (i, 0)),
            pl.BlockSpec((1, 1), lambda i: (0, 0)),
            pl.BlockSpec((1, 1), lambda i: (0, 0)),
        ],
        out_shape=[
            jax.ShapeDtypeStruct((N_ROWS, NUM_EMBEDDINGS), jnp.float32),
            jax.ShapeDtypeStruct((N_ROWS, EMBEDDING_DIM), jnp.float32),
            jax.ShapeDtypeStruct((1, 1), jnp.float32),
            jax.ShapeDtypeStruct((1, 1), jnp.float32),
        ],
        scratch_shapes=[
            pltpu.SMEM((1, 1), jnp.float32),
            pltpu.VMEM((1, NUM_EMBEDDINGS), jnp.float32),
            pltpu.VMEM((1, NUM_EMBEDDINGS), jnp.float32),
        ],
    )(flat, jnp.reshape(inputs, (16, EMBEDDING_DIM, 1024)), embedding)

    quantized_st = qst.reshape(input_shape)
    return (loss[0, 0], jnp.transpose(quantized_st, (0, 3, 1, 2)),
            perp[0, 0], enc)


# speedup vs baseline: 1.3256x; 1.3256x over previous
import jax
import jax.numpy as jnp
from jax.experimental import pallas as pl
from jax.experimental.pallas import tpu as pltpu

NUM_EMBEDDINGS = 512
EMBEDDING_DIM = 64
COMMITMENT_COST = 0.25
N_ROWS = 16 * 32 * 32
BLOCK_ROWS = 4096
GRID = N_ROWS // BLOCK_ROWS


def _vq_body(x_ref, e_ref, enc_ref, qst_ref, loss_ref, perp_ref,
             sse_acc, cnt_acc, se_scr):
    i = pl.program_id(0)
    x = x_ref[...]
    e = e_ref[...]

    @pl.when(i == 0)
    def _init():
        sse_acc[0, 0] = 0.0
        cnt_acc[...] = jnp.zeros_like(cnt_acc)

    sx = jnp.sum(x * x, axis=1, keepdims=True)
    se = jnp.sum(e * e, axis=1)
    mm2 = jax.lax.dot_general(x, e + e, (((1,), (1,)), ((), ())))

    c_iota = jax.lax.broadcasted_iota(
        jnp.int32, (mm2.shape[0], 128), 1).astype(jnp.float32)
    best = sx + se[0:128] - mm2[:, 0:128]
    bidx = c_iota
    for v in range(1, 4):
        dv = sx + se[v * 128:(v + 1) * 128] - mm2[:, v * 128:(v + 1) * 128]
        mask = dv < best
        best = jnp.where(mask, dv, best)
        bidx = jnp.where(mask, c_iota + float(v * 128), bidx)
    m = jnp.min(best, axis=1, keepdims=True)
    idxf = jnp.min(jnp.where(best == m, bidx, float(NUM_EMBEDDINGS)), axis=1,
                   keepdims=True)
    idx = idxf.astype(jnp.int32)
    iota = jax.lax.broadcasted_iota(jnp.int32, (mm2.shape[0], NUM_EMBEDDINGS), 1)
    enc = (iota == idx).astype(jnp.float32)
    enc_ref[...] = enc

    quant = jnp.dot(enc, e)
    diff = quant - x
    qst_ref[...] = x + diff

    sse_acc[0, 0] += jnp.sum(diff * diff)
    ones_row = jnp.ones((1, enc.shape[0]), jnp.float32)
    cnt_acc[...] += jax.lax.dot_general(ones_row, enc, (((1,), (0,)), ((), ())))

    @pl.when(i == GRID - 1)
    def _fini():
        mean = sse_acc[0, 0] / float(N_ROWS * EMBEDDING_DIM)
        loss_ref[...] = jnp.full((1, 1), mean + COMMITMENT_COST * mean,
                                 dtype=jnp.float32)
        p = cnt_acc[...] / float(N_ROWS)
        ent = -jnp.sum(p * jnp.log(p + 1e-10))
        perp_ref[...] = jnp.exp(jnp.full((1, 1), ent, dtype=jnp.float32))


def kernel(inputs, labels, embedding):
    del labels
    x = jnp.transpose(inputs, (0, 2, 3, 1))
    input_shape = x.shape
    flat = x.reshape(-1, EMBEDDING_DIM)

    enc, qst, loss, perp = pl.pallas_call(
        _vq_body,
        grid=(GRID,),
        in_specs=[
            pl.BlockSpec((BLOCK_ROWS, EMBEDDING_DIM), lambda i: (i, 0)),
            pl.BlockSpec((NUM_EMBEDDINGS, EMBEDDING_DIM), lambda i: (0, 0)),
        ],
        out_specs=[
            pl.BlockSpec((BLOCK_ROWS, NUM_EMBEDDINGS), lambda i: (i, 0)),
            pl.BlockSpec((BLOCK_ROWS, EMBEDDING_DIM), lambda i: (i, 0)),
            pl.BlockSpec((1, 1), lambda i: (0, 0)),
            pl.BlockSpec((1, 1), lambda i: (0, 0)),
        ],
        out_shape=[
            jax.ShapeDtypeStruct((N_ROWS, NUM_EMBEDDINGS), jnp.float32),
            jax.ShapeDtypeStruct((N_ROWS, EMBEDDING_DIM), jnp.float32),
            jax.ShapeDtypeStruct((1, 1), jnp.float32),
            jax.ShapeDtypeStruct((1, 1), jnp.float32),
        ],
        scratch_shapes=[
            pltpu.SMEM((1, 1), jnp.float32),
            pltpu.VMEM((1, NUM_EMBEDDINGS), jnp.float32),
            pltpu.VMEM((1, NUM_EMBEDDINGS), jnp.float32),
        ],
    )(flat, embedding)

    quantized_st = qst.reshape(input_shape)
    return (loss[0, 0], jnp.transpose(quantized_st, (0, 3, 1, 2)),
            perp[0, 0], enc)
